# Initial kernel scaffold; baseline (speedup 1.0000x reference)
import functools

import jax
import jax.numpy as jnp
from jax import lax
from jax.experimental import pallas as pl
from jax.experimental.pallas import tpu as pltpu
from jax.experimental.pallas import tpu_sc as plsc

N_FAMILY = 100000
DIM_FAMILY = 128
N_CHAR = 1000
DIM_CHAR = 64
BATCH = 4096
DIM_OUT = DIM_FAMILY + DIM_CHAR

NUM_CORES = 2
NUM_SUBCORES = 16
NUM_WORKERS = NUM_CORES * NUM_SUBCORES
BW = BATCH // NUM_WORKERS

_mesh = plsc.VectorSubcoreMesh(core_axis_name="c", subcore_axis_name="s")


@functools.partial(
    pl.kernel,
    mesh=_mesh,
    out_type=jax.ShapeDtypeStruct((BATCH, DIM_OUT), jnp.float32),
    scratch_types=[
        pltpu.VMEM((BW,), jnp.int32),
        pltpu.VMEM((BW,), jnp.int32),
        pltpu.VMEM((BW, DIM_FAMILY), jnp.float32),
        pltpu.VMEM((BW, DIM_CHAR), jnp.float32),
        pltpu.SemaphoreType.DMA,
        pltpu.SemaphoreType.DMA,
    ],
)
def _embed(font_hbm, char_hbm, fam_tab, chr_tab, out_hbm,
           fidx_v, cidx_v, fam_v, chr_v, sem_f, sem_c):
    wid = lax.axis_index("s") * NUM_CORES + lax.axis_index("c")
    base = wid * BW
    pltpu.sync_copy(font_hbm.at[pl.ds(base, BW)], fidx_v)
    pltpu.sync_copy(char_hbm.at[pl.ds(base, BW)], cidx_v)
    cp_f = pltpu.async_copy(fam_tab.at[fidx_v], fam_v, sem_f)
    cp_c = pltpu.async_copy(chr_tab.at[cidx_v], chr_v, sem_c)
    cp_f.wait()
    cp_c.wait()
    pltpu.sync_copy(fam_v, out_hbm.at[pl.ds(base, BW), pl.ds(0, DIM_FAMILY)])
    pltpu.sync_copy(chr_v, out_hbm.at[pl.ds(base, BW), pl.ds(DIM_FAMILY, DIM_CHAR)])


def kernel(font_idx, char_idx, family_table, char_table):
    return _embed(font_idx.astype(jnp.int32), char_idx.astype(jnp.int32),
                  family_table, char_table)


# speedup vs baseline: 1.4087x; 1.4087x over previous
import functools

import jax
import jax.numpy as jnp
from jax import lax
from jax.experimental import pallas as pl
from jax.experimental.pallas import tpu as pltpu
from jax.experimental.pallas import tpu_sc as plsc

N_FAMILY = 100000
DIM_FAMILY = 128
N_CHAR = 1000
DIM_CHAR = 64
BATCH = 4096
DIM_OUT = DIM_FAMILY + DIM_CHAR

NUM_CORES = 2
NUM_SUBCORES = 16
NUM_WORKERS = NUM_CORES * NUM_SUBCORES
BW = BATCH // NUM_WORKERS

_mesh = plsc.VectorSubcoreMesh(core_axis_name="c", subcore_axis_name="s")


@functools.partial(
    pl.kernel,
    mesh=_mesh,
    compiler_params=pltpu.CompilerParams(use_tc_tiling_on_sc=False),
    out_type=jax.ShapeDtypeStruct((BATCH, DIM_OUT), jnp.float32),
    scratch_types=[
        pltpu.VMEM((BW,), jnp.int32),
        pltpu.VMEM((BW,), jnp.int32),
        pltpu.VMEM((BW, DIM_FAMILY), jnp.float32),
        pltpu.VMEM((BW, DIM_CHAR), jnp.float32),
        pltpu.SemaphoreType.DMA,
        pltpu.SemaphoreType.DMA,
    ],
)
def _embed(font_hbm, char_hbm, fam_tab, chr_tab, out_hbm,
           fidx_v, cidx_v, fam_v, chr_v, sem_f, sem_c):
    wid = lax.axis_index("s") * NUM_CORES + lax.axis_index("c")
    base = wid * BW
    pltpu.sync_copy(font_hbm.at[pl.ds(base, BW)], fidx_v)
    pltpu.sync_copy(char_hbm.at[pl.ds(base, BW)], cidx_v)
    cp_f = pltpu.async_copy(fam_tab.at[fidx_v], fam_v, sem_f)
    cp_c = pltpu.async_copy(chr_tab.at[cidx_v], chr_v, sem_c)
    cp_f.wait()
    cp_c.wait()
    pltpu.sync_copy(fam_v, out_hbm.at[pl.ds(base, BW), pl.ds(0, DIM_FAMILY)])
    pltpu.sync_copy(chr_v, out_hbm.at[pl.ds(base, BW), pl.ds(DIM_FAMILY, DIM_CHAR)])


def kernel(font_idx, char_idx, family_table, char_table):
    return _embed(font_idx.astype(jnp.int32), char_idx.astype(jnp.int32),
                  family_table, char_table)


# speedup vs baseline: 1.4351x; 1.0187x over previous
import functools

import jax
import jax.numpy as jnp
from jax import lax
from jax.experimental import pallas as pl
from jax.experimental.pallas import tpu as pltpu
from jax.experimental.pallas import tpu_sc as plsc

N_FAMILY = 100000
DIM_FAMILY = 128
N_CHAR = 1000
DIM_CHAR = 64
BATCH = 4096
DIM_OUT = DIM_FAMILY + DIM_CHAR

NUM_CORES = 2
NUM_SUBCORES = 16
NUM_WORKERS = NUM_CORES * NUM_SUBCORES
BW = BATCH // NUM_WORKERS

_mesh = plsc.VectorSubcoreMesh(core_axis_name="c", subcore_axis_name="s")


@functools.partial(
    pl.kernel,
    mesh=_mesh,
    compiler_params=pltpu.CompilerParams(use_tc_tiling_on_sc=False),
    out_type=jax.ShapeDtypeStruct((BATCH, DIM_OUT), jnp.float32),
    scratch_types=[
        pltpu.VMEM((BW,), jnp.int32),
        pltpu.VMEM((BW,), jnp.int32),
        pltpu.VMEM((BW, DIM_FAMILY), jnp.float32),
        pltpu.VMEM((BW, DIM_CHAR), jnp.float32),
        pltpu.SemaphoreType.DMA,
        pltpu.SemaphoreType.DMA,
        pltpu.SemaphoreType.DMA,
        pltpu.SemaphoreType.DMA,
    ],
)
def _embed(font_hbm, char_hbm, fam_tab, chr_tab, out_hbm,
           fidx_v, cidx_v, fam_v, chr_v, sem_i, sem_f, sem_c, sem_o):
    wid = lax.axis_index("s") * NUM_CORES + lax.axis_index("c")
    base = wid * BW
    cp_fi = pltpu.async_copy(font_hbm.at[pl.ds(base, BW)], fidx_v, sem_i)
    cp_ci = pltpu.async_copy(char_hbm.at[pl.ds(base, BW)], cidx_v, sem_i)
    cp_fi.wait()
    cp_f = pltpu.async_copy(fam_tab.at[fidx_v], fam_v, sem_f)
    cp_ci.wait()
    cp_c = pltpu.async_copy(chr_tab.at[cidx_v], chr_v, sem_c)
    cp_f.wait()
    cp_of = pltpu.async_copy(fam_v, out_hbm.at[pl.ds(base, BW), pl.ds(0, DIM_FAMILY)], sem_o)
    cp_c.wait()
    cp_oc = pltpu.async_copy(chr_v, out_hbm.at[pl.ds(base, BW), pl.ds(DIM_FAMILY, DIM_CHAR)], sem_o)
    cp_of.wait()
    cp_oc.wait()


def kernel(font_idx, char_idx, family_table, char_table):
    return _embed(font_idx.astype(jnp.int32), char_idx.astype(jnp.int32),
                  family_table, char_table)
